# baseline (speedup 1.0000x reference)
import functools

import jax
import jax.numpy as jnp
from jax import lax
from jax.experimental import pallas as pl
from jax.experimental.pallas import tpu as pltpu
from jax.experimental.pallas import tpu_sc as plsc

NUM_DEVICES = 8
NUM_EXPERTS = 16
TOP_K = 2
NC, NS, L = 2, 16, 16
NW = NC * NS
T = 4096
D_MODEL = 1024
S = T * TOP_K
SLOTS_PER_W = S // NW
BSLOTS = 32
NZCH = SLOTS_PER_W // BSLOTS
BWORDS = BSLOTS * D_MODEL
TOKROWS = BSLOTS // TOP_K
TWORDS = TOKROWS * D_MODEL
NTB = 3
SEG = 4 * L


def _copy_row(dst_ref, dst_w, src_ref, src_w):
    def body(seg, _):
        o = seg * SEG
        for q in range(4):
            dst_ref[pl.ds(dst_w + o + q * L, L)] = (
                src_ref[pl.ds(src_w + o + q * L, L)])
        return 0
    lax.fori_loop(0, D_MODEL // SEG, body, 0)


def _zero_row(dst_ref, dst_w):
    z16 = jnp.zeros((L,), jnp.float32)
    def body(seg, _):
        o = seg * SEG
        for q in range(4):
            dst_ref[pl.ds(dst_w + o + q * L, L)] = z16
        return 0
    lax.fori_loop(0, D_MODEL // SEG, body, 0)


def _walk(dev_v, z, d, lane, buf, tb):
    for g in range(BSLOTS // L):
        dev16 = dev_v[pl.ds((z * (BSLOTS // L) + g) * L, L)]
        mask = dev16 == d
        mi = mask.astype(jnp.int32)
        prefix = jnp.cumsum(mi)
        cnt = jnp.sum(mi)

        def body(k, _):
            onehot = mask & (prefix == (k + 1))
            r = g * L + jnp.sum(jnp.where(onehot, lane, 0))
            if tb is None:
                _zero_row(buf, r * D_MODEL)
            else:
                _copy_row(buf, r * D_MODEL, tb,
                          lax.shift_right_logical(r, 1) * D_MODEL)
            return 0
        lax.fori_loop(0, cnt, body, 0)


def _sc_dispatch(in_hbm, idx_hbm, map_hbm, out_hbm,
                 ba, bb, t0, t1, t2, idx_v, map_v, dev_v,
                 sa, sb, tsem):
    wid = lax.axis_index("s") * NC + lax.axis_index("c")
    base = pl.multiple_of(wid * SLOTS_PER_W, SLOTS_PER_W)
    tw_base = pl.multiple_of(base * (D_MODEL // TOP_K), TWORDS)
    tbufs = [t0, t1, t2]
    lane = jnp.arange(L, dtype=jnp.int32)

    def stream(buf, sem, z, d):
        off = pl.multiple_of((d * S + base + z * BSLOTS) * D_MODEL, BWORDS)
        cp = pltpu.make_async_copy(buf, out_hbm.at[pl.ds(off, BWORDS)], sem)
        cp.start()

    def drain(buf, sem):
        pltpu.make_async_copy(buf, out_hbm.at[pl.ds(0, BWORDS)], sem).wait()

    pltpu.sync_copy(idx_hbm.at[pl.ds(base, SLOTS_PER_W)], idx_v)
    pltpu.sync_copy(map_hbm, map_v)
    for c in range(SLOTS_PER_W // L):
        i16 = idx_v[pl.ds(c * L, L)]
        dev_v[pl.ds(c * L, L)] = plsc.load_gather(map_v, [i16])

    pend_t = {}
    for z in range(min(NTB, NZCH)):
        t = pltpu.make_async_copy(
            in_hbm.at[pl.ds(tw_base + z * TWORDS, TWORDS)],
            tbufs[z % NTB], tsem)
        t.start()
        pend_t[z] = t
    def zero_all(r, _, buf=None):
        _zero_row(buf, r * D_MODEL)
        return 0
    lax.fori_loop(0, BSLOTS, functools.partial(zero_all, buf=ba), 0)
    lax.fori_loop(0, BSLOTS, functools.partial(zero_all, buf=bb), 0)

    for z in range(NZCH):
        tb = tbufs[z % NTB]
        pend_t[z].wait()
        if z > 0:
            drain(ba, sa)
            _walk(dev_v, z - 1, NUM_DEVICES - 2, lane, ba, None)
            drain(bb, sb)
            _walk(dev_v, z - 1, NUM_DEVICES - 1, lane, bb, None)
        _walk(dev_v, z, 0, lane, ba, tb)
        stream(ba, sa, z, 0)
        _walk(dev_v, z, 1, lane, bb, tb)
        stream(bb, sb, z, 1)

        def pair(dp, _, tb=tb, z=z):
            d = 2 * dp
            drain(ba, sa)
            _walk(dev_v, z, d - 2, lane, ba, None)
            _walk(dev_v, z, d, lane, ba, tb)
            stream(ba, sa, z, d)
            drain(bb, sb)
            _walk(dev_v, z, d - 1, lane, bb, None)
            _walk(dev_v, z, d + 1, lane, bb, tb)
            stream(bb, sb, z, d + 1)
            return 0
        lax.fori_loop(1, NUM_DEVICES // 2, pair, 0)

        if z + NTB < NZCH:
            t = pltpu.make_async_copy(
                in_hbm.at[pl.ds(tw_base + (z + NTB) * TWORDS, TWORDS)],
                tbufs[z % NTB], tsem)
            t.start()
            pend_t[z + NTB] = t
    drain(ba, sa)
    drain(bb, sb)


def kernel(input_tensor, expert_indices, expert_mapping):
    idx_flat = expert_indices.reshape(-1)
    in_flat = input_tensor.reshape(-1)
    mesh = plsc.VectorSubcoreMesh(core_axis_name="c", subcore_axis_name="s")
    k = functools.partial(
        pl.kernel,
        out_type=jax.ShapeDtypeStruct((NUM_DEVICES * S * D_MODEL,),
                                      jnp.float32),
        mesh=mesh,
        compiler_params=pltpu.CompilerParams(needs_layout_passes=False),
        scratch_types=[
            pltpu.VMEM((BWORDS,), jnp.float32),
            pltpu.VMEM((BWORDS,), jnp.float32),
            pltpu.VMEM((TWORDS,), jnp.float32),
            pltpu.VMEM((TWORDS,), jnp.float32),
            pltpu.VMEM((TWORDS,), jnp.float32),
            pltpu.VMEM((SLOTS_PER_W,), jnp.int32),
            pltpu.VMEM((NUM_EXPERTS,), jnp.int32),
            pltpu.VMEM((SLOTS_PER_W,), jnp.int32),
            pltpu.SemaphoreType.DMA,
            pltpu.SemaphoreType.DMA,
            pltpu.SemaphoreType.DMA,
        ],
    )(_sc_dispatch)
    out = k(in_flat, idx_flat, expert_mapping)
    return out.reshape(NUM_DEVICES, S, D_MODEL)


# speedup vs baseline: 3.4276x; 3.4276x over previous
import functools

import jax
import jax.numpy as jnp
from jax import lax
from jax.experimental import pallas as pl
from jax.experimental.pallas import tpu as pltpu
from jax.experimental.pallas import tpu_sc as plsc

NUM_DEVICES = 8
NUM_EXPERTS = 16
TOP_K = 2
NC, NS, L = 2, 16, 16
NW = NC * NS
T = 4096
D_MODEL = 1024
S = T * TOP_K
SLOTS_PER_W = S // NW
ZROWS = 64
NZ = (NUM_DEVICES * SLOTS_PER_W) // ZROWS
NCHUNK = SLOTS_PER_W // L
TROWS = L // TOP_K
NBUF = 7


def _sc_dispatch(in_hbm, idx_hbm, map_hbm, zsrc_hbm, out_hbm,
                 zbuf, idx_v, map_v, rowid_v,
                 tb0, tb1, tb2, tb3, tb4, tb5, tb6,
                 zsem, tsems, ssems):
    wid = lax.axis_index("s") * NC + lax.axis_index("c")
    base = pl.multiple_of(wid * SLOTS_PER_W, SLOTS_PER_W)
    tbase = pl.multiple_of(base // TOP_K, SLOTS_PER_W // TOP_K)
    tbufs = [tb0, tb1, tb2, tb3, tb4, tb5, tb6]

    zstage = pltpu.make_async_copy(zsrc_hbm, zbuf, zsem)
    zstage.start()
    zstage.wait()
    zcps = []
    for d in range(NUM_DEVICES):
        plane_base = d * S + base
        for z in range(SLOTS_PER_W // ZROWS):
            cp = pltpu.make_async_copy(
                zbuf,
                out_hbm.at[pl.ds(plane_base + z * ZROWS, ZROWS), :],
                zsem,
            )
            cp.start()
            zcps.append(cp)

    pend_t = {}
    for c in range(min(NBUF, NCHUNK)):
        t = pltpu.make_async_copy(
            in_hbm.at[pl.ds(tbase + c * TROWS, TROWS), :],
            tbufs[c % NBUF], tsems.at[c % NBUF])
        t.start()
        pend_t[c] = t

    pltpu.sync_copy(idx_hbm.at[pl.ds(base, SLOTS_PER_W)], idx_v)
    pltpu.sync_copy(map_hbm, map_v)
    lane = jnp.arange(L, dtype=jnp.int32)
    for c in range(NCHUNK):
        i16 = idx_v[pl.ds(c * L, L)]
        dev16 = plsc.load_gather(map_v, [i16])
        slot16 = base + c * L + lane
        row16 = dev16 * S + slot16
        plsc.store_scatter(
            rowid_v,
            [2 * c + (lane & 1), lax.shift_right_logical(lane, 1)],
            row16,
        )

    for cp in zcps:
        cp.wait()

    for c in range(NCHUNK):
        b = c % NBUF
        pend_t[c].wait()
        s_ev = pltpu.make_async_copy(
            tbufs[b], out_hbm.at[rowid_v.at[2 * c]], ssems.at[b])
        s_od = pltpu.make_async_copy(
            tbufs[b], out_hbm.at[rowid_v.at[2 * c + 1]], ssems.at[b])
        s_ev.start()
        s_od.start()
        if c + NBUF < NCHUNK:
            s_ev.wait()
            s_od.wait()
            t = pltpu.make_async_copy(
                in_hbm.at[pl.ds(tbase + (c + NBUF) * TROWS, TROWS), :],
                tbufs[b], tsems.at[b])
            t.start()
            pend_t[c + NBUF] = t
        else:
            pend_t[c] = (s_ev, s_od)
    for c in range(NCHUNK - NBUF, NCHUNK):
        if c >= 0:
            s_ev, s_od = pend_t[c]
            s_ev.wait()
            s_od.wait()


def kernel(input_tensor, expert_indices, expert_mapping):
    idx_flat = expert_indices.reshape(-1)
    zsrc = jnp.zeros((ZROWS, D_MODEL), jnp.float32)
    mesh = plsc.VectorSubcoreMesh(core_axis_name="c", subcore_axis_name="s")
    k = functools.partial(
        pl.kernel,
        out_type=jax.ShapeDtypeStruct((NUM_DEVICES * S, D_MODEL), jnp.float32),
        mesh=mesh,
        compiler_params=pltpu.CompilerParams(needs_layout_passes=False),
        scratch_types=[
            pltpu.VMEM((ZROWS, D_MODEL), jnp.float32),
            pltpu.VMEM((SLOTS_PER_W,), jnp.int32),
            pltpu.VMEM((NUM_EXPERTS,), jnp.int32),
            pltpu.VMEM((2 * NCHUNK, TROWS), jnp.int32),
        ] + [pltpu.VMEM((TROWS, D_MODEL), jnp.float32)] * NBUF + [
            pltpu.SemaphoreType.DMA,
            pltpu.SemaphoreType.DMA((NBUF,)),
            pltpu.SemaphoreType.DMA((NBUF,)),
        ],
    )(_sc_dispatch)
    out = k(input_tensor, idx_flat, expert_mapping, zsrc)
    return out.reshape(NUM_DEVICES, S, D_MODEL)


# speedup vs baseline: 3.4751x; 1.0138x over previous
import functools

import jax
import jax.numpy as jnp
from jax import lax
from jax.experimental import pallas as pl
from jax.experimental.pallas import tpu as pltpu
from jax.experimental.pallas import tpu_sc as plsc

NUM_DEVICES = 8
NUM_EXPERTS = 16
TOP_K = 2
NC, NS, L = 2, 16, 16
NW = NC * NS
T = 4096
D_MODEL = 1024
S = T * TOP_K
SLOTS_PER_W = S // NW
ZROWS = 64
NZ = (NUM_DEVICES * SLOTS_PER_W) // ZROWS
NCHUNK = SLOTS_PER_W // L
TROWS = L // TOP_K
NBUF = 7


def _sc_dispatch(in_hbm, idx_hbm, map_hbm, zsrc_hbm, out_hbm,
                 zbuf, idx_v, map_v, rowid_v,
                 tb0, tb1, tb2, tb3, tb4, tb5, tb6,
                 zsem, tsems, ssems):
    wid = lax.axis_index("s") * NC + lax.axis_index("c")
    base = pl.multiple_of(wid * SLOTS_PER_W, SLOTS_PER_W)
    tbase = pl.multiple_of(base // TOP_K, SLOTS_PER_W // TOP_K)
    tbufs = [tb0, tb1, tb2, tb3, tb4, tb5, tb6]

    zstage = pltpu.make_async_copy(zsrc_hbm, zbuf, zsem)
    zstage.start()
    pltpu.sync_copy(idx_hbm.at[pl.ds(base, SLOTS_PER_W)], idx_v)
    pltpu.sync_copy(map_hbm, map_v)

    lane = jnp.arange(L, dtype=jnp.int32)
    for c in range(NCHUNK):
        i16 = idx_v[pl.ds(c * L, L)]
        dev16 = plsc.load_gather(map_v, [i16])
        slot16 = base + c * L + lane
        row16 = dev16 * S + slot16
        plsc.store_scatter(
            rowid_v,
            [2 * c + (lane & 1), lax.shift_right_logical(lane, 1)],
            row16,
        )
    zstage.wait()

    zcps = []
    for d in range(NUM_DEVICES):
        plane_base = d * S + base
        for z in range(SLOTS_PER_W // ZROWS):
            cp = pltpu.make_async_copy(
                zbuf,
                out_hbm.at[pl.ds(plane_base + z * ZROWS, ZROWS), :],
                zsem,
            )
            cp.start()
            zcps.append(cp)

    pend_t = {}
    for c in range(min(NBUF, NCHUNK)):
        t = pltpu.make_async_copy(
            in_hbm.at[pl.ds(tbase + c * TROWS, TROWS), :],
            tbufs[c % NBUF], tsems.at[c % NBUF])
        t.start()
        pend_t[c] = t

    for cp in zcps:
        cp.wait()

    for c in range(NCHUNK):
        b = c % NBUF
        pend_t[c].wait()
        s_ev = pltpu.make_async_copy(
            tbufs[b], out_hbm.at[rowid_v.at[2 * c]], ssems.at[b])
        s_od = pltpu.make_async_copy(
            tbufs[b], out_hbm.at[rowid_v.at[2 * c + 1]], ssems.at[b])
        s_ev.start()
        s_od.start()
        if c + NBUF < NCHUNK:
            s_ev.wait()
            s_od.wait()
            t = pltpu.make_async_copy(
                in_hbm.at[pl.ds(tbase + (c + NBUF) * TROWS, TROWS), :],
                tbufs[b], tsems.at[b])
            t.start()
            pend_t[c + NBUF] = t
        else:
            pend_t[c] = (s_ev, s_od)
    for c in range(NCHUNK - NBUF, NCHUNK):
        if c >= 0:
            s_ev, s_od = pend_t[c]
            s_ev.wait()
            s_od.wait()


def kernel(input_tensor, expert_indices, expert_mapping):
    idx_flat = expert_indices.reshape(-1)
    zsrc = jnp.zeros((ZROWS, D_MODEL), jnp.float32)
    mesh = plsc.VectorSubcoreMesh(core_axis_name="c", subcore_axis_name="s")
    k = functools.partial(
        pl.kernel,
        out_type=jax.ShapeDtypeStruct((NUM_DEVICES * S, D_MODEL), jnp.float32),
        mesh=mesh,
        compiler_params=pltpu.CompilerParams(needs_layout_passes=False),
        scratch_types=[
            pltpu.VMEM((ZROWS, D_MODEL), jnp.float32),
            pltpu.VMEM((SLOTS_PER_W,), jnp.int32),
            pltpu.VMEM((NUM_EXPERTS,), jnp.int32),
            pltpu.VMEM((2 * NCHUNK, TROWS), jnp.int32),
        ] + [pltpu.VMEM((TROWS, D_MODEL), jnp.float32)] * NBUF + [
            pltpu.SemaphoreType.DMA,
            pltpu.SemaphoreType.DMA((NBUF,)),
            pltpu.SemaphoreType.DMA((NBUF,)),
        ],
    )(_sc_dispatch)
    out = k(input_tensor, idx_flat, expert_mapping, zsrc)
    return out.reshape(NUM_DEVICES, S, D_MODEL)


# speedup vs baseline: 3.5703x; 1.0274x over previous
import functools

import jax
import jax.numpy as jnp
from jax import lax
from jax.experimental import pallas as pl
from jax.experimental.pallas import tpu as pltpu
from jax.experimental.pallas import tpu_sc as plsc

NUM_DEVICES = 8
NUM_EXPERTS = 16
TOP_K = 2
NC, NS, L = 2, 16, 16
NW = NC * NS
T = 4096
D_MODEL = 1024
S = T * TOP_K
SLOTS_PER_W = S // NW
ZROWS = 32
NZ = (NUM_DEVICES * SLOTS_PER_W) // ZROWS
NCHUNK = SLOTS_PER_W // L
TROWS = L // TOP_K
NBUF = 11


def _sc_dispatch(in_hbm, idx_hbm, map_hbm, zsrc_hbm, out_hbm,
                 zbuf, idx_v, map_v, rowid_v,
                 tb0, tb1, tb2, tb3, tb4, tb5, tb6, tb7, tb8, tb9, tb10,
                 zsem, tsems, ssems):
    wid = lax.axis_index("s") * NC + lax.axis_index("c")
    base = pl.multiple_of(wid * SLOTS_PER_W, SLOTS_PER_W)
    tbase = pl.multiple_of(base // TOP_K, SLOTS_PER_W // TOP_K)
    tbufs = [tb0, tb1, tb2, tb3, tb4, tb5, tb6, tb7, tb8, tb9, tb10]

    zstage = pltpu.make_async_copy(zsrc_hbm, zbuf, zsem)
    zstage.start()
    pltpu.sync_copy(idx_hbm.at[pl.ds(base, SLOTS_PER_W)], idx_v)
    pltpu.sync_copy(map_hbm, map_v)

    lane = jnp.arange(L, dtype=jnp.int32)
    for c in range(NCHUNK):
        i16 = idx_v[pl.ds(c * L, L)]
        dev16 = plsc.load_gather(map_v, [i16])
        slot16 = base + c * L + lane
        row16 = dev16 * S + slot16
        plsc.store_scatter(
            rowid_v,
            [2 * c + (lane & 1), lax.shift_right_logical(lane, 1)],
            row16,
        )
    zstage.wait()

    zcps = []
    for d in range(NUM_DEVICES):
        plane_base = d * S + base
        for z in range(SLOTS_PER_W // ZROWS):
            cp = pltpu.make_async_copy(
                zbuf,
                out_hbm.at[pl.ds(plane_base + z * ZROWS, ZROWS), :],
                zsem,
            )
            cp.start()
            zcps.append(cp)

    pend_t = {}
    for c in range(min(NBUF, NCHUNK)):
        t = pltpu.make_async_copy(
            in_hbm.at[pl.ds(tbase + c * TROWS, TROWS), :],
            tbufs[c % NBUF], tsems.at[c % NBUF])
        t.start()
        pend_t[c] = t

    for cp in zcps:
        cp.wait()

    for c in range(NCHUNK):
        b = c % NBUF
        pend_t[c].wait()
        s_ev = pltpu.make_async_copy(
            tbufs[b], out_hbm.at[rowid_v.at[2 * c]], ssems.at[b])
        s_od = pltpu.make_async_copy(
            tbufs[b], out_hbm.at[rowid_v.at[2 * c + 1]], ssems.at[b])
        s_ev.start()
        s_od.start()
        if c + NBUF < NCHUNK:
            s_ev.wait()
            s_od.wait()
            t = pltpu.make_async_copy(
                in_hbm.at[pl.ds(tbase + (c + NBUF) * TROWS, TROWS), :],
                tbufs[b], tsems.at[b])
            t.start()
            pend_t[c + NBUF] = t
        else:
            pend_t[c] = (s_ev, s_od)
    for c in range(NCHUNK - NBUF, NCHUNK):
        if c >= 0:
            s_ev, s_od = pend_t[c]
            s_ev.wait()
            s_od.wait()


def kernel(input_tensor, expert_indices, expert_mapping):
    idx_flat = expert_indices.reshape(-1)
    zsrc = jnp.zeros((ZROWS, D_MODEL), jnp.float32)
    mesh = plsc.VectorSubcoreMesh(core_axis_name="c", subcore_axis_name="s")
    k = functools.partial(
        pl.kernel,
        out_type=jax.ShapeDtypeStruct((NUM_DEVICES * S, D_MODEL), jnp.float32),
        mesh=mesh,
        compiler_params=pltpu.CompilerParams(needs_layout_passes=False),
        scratch_types=[
            pltpu.VMEM((ZROWS, D_MODEL), jnp.float32),
            pltpu.VMEM((SLOTS_PER_W,), jnp.int32),
            pltpu.VMEM((NUM_EXPERTS,), jnp.int32),
            pltpu.VMEM((2 * NCHUNK, TROWS), jnp.int32),
        ] + [pltpu.VMEM((TROWS, D_MODEL), jnp.float32)] * NBUF + [
            pltpu.SemaphoreType.DMA,
            pltpu.SemaphoreType.DMA((NBUF,)),
            pltpu.SemaphoreType.DMA((NBUF,)),
        ],
    )(_sc_dispatch)
    out = k(input_tensor, idx_flat, expert_mapping, zsrc)
    return out.reshape(NUM_DEVICES, S, D_MODEL)
